# baseline (speedup 1.0000x reference)
import functools

import jax
import jax.numpy as jnp
from jax.experimental import pallas as pl
from jax.experimental.pallas import tpu as pltpu


def _fused_lowrank_body(x_ref, w1t_ref, w2t_ref, b2_ref, o_ref):
    h = jnp.dot(x_ref[...], w1t_ref[...], preferred_element_type=jnp.float32)
    y = jnp.dot(h.astype(w2t_ref.dtype), w2t_ref[...],
                preferred_element_type=jnp.float32)
    o_ref[...] = (y + b2_ref[...]).astype(o_ref.dtype)


@functools.partial(jax.jit, static_argnames=("tm",))
def _lowrank_call(x, w1t, w2t, b2p, tm):
    B, d_in = x.shape
    rank_p = w1t.shape[1]
    d_out_p = w2t.shape[1]
    grid = pl.cdiv(B, tm)
    return pl.pallas_call(
        _fused_lowrank_body,
        out_shape=jax.ShapeDtypeStruct((B, d_out_p), jnp.bfloat16),
        grid=(grid,),
        in_specs=[
            pl.BlockSpec((tm, d_in), lambda i: (i, 0)),
            pl.BlockSpec((d_in, rank_p), lambda i: (0, 0)),
            pl.BlockSpec((rank_p, d_out_p), lambda i: (0, 0)),
            pl.BlockSpec((1, d_out_p), lambda i: (0, 0)),
        ],
        out_specs=pl.BlockSpec((tm, d_out_p), lambda i: (i, 0)),
        compiler_params=pltpu.CompilerParams(
            dimension_semantics=("parallel",),
            vmem_limit_bytes=100 * 1024 * 1024,
        ),
    )(x, w1t, w2t, b2p)


def kernel(x, w1t, w2t, b2p):
    B = x.shape[0]
    tm = 1024
    while tm > 8 and B % tm:
        tm //= 2
    x = x if x.dtype == w1t.dtype else x.astype(w1t.dtype)
    return _lowrank_call(x, w1t, w2t, b2p, max(tm, 8))


# speedup vs baseline: 1.0013x; 1.0013x over previous
import functools

import jax
import jax.numpy as jnp
from jax.experimental import pallas as pl
from jax.experimental.pallas import tpu as pltpu


def _fused_lowrank_body(x_ref, w1t_ref, w2t_ref, b2_ref, o_ref):
    h = jnp.dot(x_ref[...], w1t_ref[...], preferred_element_type=jnp.float32)
    y = jnp.dot(h.astype(w2t_ref.dtype), w2t_ref[...],
                preferred_element_type=jnp.float32)
    o_ref[...] = (y + b2_ref[...]).astype(o_ref.dtype)


@functools.partial(jax.jit, static_argnames=("tm",))
def _lowrank_call(x, w1t, w2t, b2p, tm):
    B, d_in = x.shape
    rank_p = w1t.shape[1]
    d_out_p = w2t.shape[1]
    grid = pl.cdiv(B, tm)
    return pl.pallas_call(
        _fused_lowrank_body,
        out_shape=jax.ShapeDtypeStruct((B, d_out_p), jnp.bfloat16),
        grid=(grid,),
        in_specs=[
            pl.BlockSpec((tm, d_in), lambda i: (i, 0)),
            pl.BlockSpec((d_in, rank_p), lambda i: (0, 0)),
            pl.BlockSpec((rank_p, d_out_p), lambda i: (0, 0)),
            pl.BlockSpec((1, d_out_p), lambda i: (0, 0)),
        ],
        out_specs=pl.BlockSpec((tm, d_out_p), lambda i: (i, 0)),
        compiler_params=pltpu.CompilerParams(
            dimension_semantics=("parallel",),
            vmem_limit_bytes=100 * 1024 * 1024,
        ),
    )(x, w1t, w2t, b2p)


def kernel(x, w1t, w2t, b2p):
    B = x.shape[0]
    tm = 512
    while tm > 8 and B % tm:
        tm //= 2
    x = x if x.dtype == w1t.dtype else x.astype(w1t.dtype)
    return _lowrank_call(x, w1t, w2t, b2p, max(tm, 8))
